# baseline (speedup 1.0000x reference)
import jax
import jax.numpy as jnp
import numpy as np
from jax.experimental import pallas as pl
from jax.experimental.pallas import tpu as pltpu

B = 64
N = 32
R = 16
F = 16
G = 4960

OCT = 8
NOCT = B // OCT


def _fused_body(x_ref, dg_ref, w_ref, wd_ref, o_ref, scr_ref, dscr_ref, out_scr):
    y = jnp.dot(x_ref[0], w_ref[...], preferred_element_type=jnp.float32)
    scr_ref[0] = y[:, 0:128]
    scr_ref[1] = y[:, 128:256]
    scr_ref[2] = y[:, 256:384]
    d = jnp.dot(dg_ref[0], wd_ref[...], preferred_element_type=jnp.float32)
    dscr_ref[0] = d[:, 0:128]
    dscr_ref[1] = d[:, 128:256]
    dscr_ref[2] = d[:, 256:384]
    off = 0
    for a in range(N - 2):
        for b2 in range(a + 1, N - 1):
            L = (N - 1) - b2
            rowv = scr_ref[0, a * N + b2, :] + dscr_ref[0, a, :] + dscr_ref[1, b2, :]
            s02 = scr_ref[1, pl.ds(a * N + b2 + 1, L), :]
            s12 = scr_ref[2, pl.ds(b2 * N + b2 + 1, L), :]
            d2 = dscr_ref[2, pl.ds(b2 + 1, L), :]
            out_scr[pl.ds(off, L), :] = rowv[None, :] + s02 + s12 + d2
            off += L
    for bi in range(OCT):
        o_ref[bi] = out_scr[:, pl.ds(bi * F, F)]


def _fused_tc(xab, dg8, w8, wd8):
    return pl.pallas_call(
        _fused_body,
        grid=(NOCT,),
        in_specs=[
            pl.BlockSpec((1, N * N, 2 * R * OCT), lambda i: (i, 0, 0)),
            pl.BlockSpec((1, N, R * OCT), lambda i: (i, 0, 0)),
            pl.BlockSpec((2 * R * OCT, 3 * OCT * F), lambda i: (0, 0)),
            pl.BlockSpec((R * OCT, 3 * OCT * F), lambda i: (0, 0)),
        ],
        out_specs=pl.BlockSpec((OCT, G, F), lambda i: (i, 0, 0)),
        out_shape=jax.ShapeDtypeStruct((B, G, F), jnp.float32),
        scratch_shapes=[
            pltpu.VMEM((3, N * N, OCT * F), jnp.float32),
            pltpu.VMEM((3, N, OCT * F), jnp.float32),
            pltpu.VMEM((G, OCT * F), jnp.float32),
        ],
        compiler_params=pltpu.CompilerParams(
            dimension_semantics=("parallel",),
        ),
    )(xab, dg8, w8, wd8)


def kernel(inputs, filters):
    idx = jnp.arange(N)
    in_t = jnp.swapaxes(inputs, 1, 2)
    comp = jnp.concatenate([inputs, in_t], axis=-1)
    xab = (
        comp.reshape(NOCT, OCT, N * N, 2 * R)
        .transpose(0, 2, 3, 1)
        .reshape(NOCT, N * N, 2 * R * OCT)
    )
    diag = inputs[:, idx, idx, :]
    dg8 = (
        diag.reshape(NOCT, OCT, N, R)
        .transpose(0, 2, 1, 3)
        .reshape(NOCT, N, OCT * R)
    )

    def fpq(p, q):
        return filters[:, p, q, :].T

    eye = jnp.eye(OCT, dtype=jnp.float32)
    wuv = jnp.stack([fpq(0, 1), fpq(0, 2), fpq(1, 2)], axis=1)
    wvu = jnp.stack([fpq(1, 0), fpq(2, 0), fpq(2, 1)], axis=1)
    wa3 = jnp.concatenate([wuv, wvu], axis=0)
    w8 = jnp.einsum("kcf,bj->kbcjf", wa3, eye).reshape(2 * R * OCT, 3 * OCT * F)
    wdg = jnp.stack([fpq(0, 0), fpq(1, 1), fpq(2, 2)], axis=1)
    wd8 = jnp.einsum("rcf,bj->brcjf", wdg, eye).reshape(R * OCT, 3 * OCT * F)

    return _fused_tc(xab, dg8, w8, wd8)


# speedup vs baseline: 1.7500x; 1.7500x over previous
import jax
import jax.numpy as jnp
import numpy as np
from jax.experimental import pallas as pl
from jax.experimental.pallas import tpu as pltpu

B = 64
N = 32
R = 16
F = 16
G = 4960

OCT = 8
NOCT = B // OCT


def _fused_body(x_ref, dg_ref, w_ref, wd_ref, o_ref, scr_ref, dscr_ref):
    y = jnp.dot(x_ref[0], w_ref[...], preferred_element_type=jnp.float32)
    scr_ref[0] = y[:, 0:128]
    scr_ref[1] = y[:, 128:256]
    scr_ref[2] = y[:, 256:384]
    d = jnp.dot(dg_ref[0], wd_ref[...], preferred_element_type=jnp.float32)
    dscr_ref[0] = d[:, 0:128]
    dscr_ref[1] = d[:, 128:256]
    dscr_ref[2] = d[:, 256:384]
    off = 0
    for a in range(N - 2):
        for b2 in range(a + 1, N - 1):
            L = (N - 1) - b2
            rowv = scr_ref[0, a * N + b2, :] + dscr_ref[0, a, :] + dscr_ref[1, b2, :]
            s02 = scr_ref[1, pl.ds(a * N + b2 + 1, L), :]
            s12 = scr_ref[2, pl.ds(b2 * N + b2 + 1, L), :]
            d2 = dscr_ref[2, pl.ds(b2 + 1, L), :]
            o_ref[0, pl.ds(off, L), :] = rowv[None, :] + s02 + s12 + d2
            off += L


def _fused_tc(xab, dg8, w8, wd8):
    return pl.pallas_call(
        _fused_body,
        grid=(NOCT,),
        in_specs=[
            pl.BlockSpec((1, N * N, 2 * R * OCT), lambda i: (i, 0, 0)),
            pl.BlockSpec((1, N, R * OCT), lambda i: (i, 0, 0)),
            pl.BlockSpec((2 * R * OCT, 3 * OCT * F), lambda i: (0, 0)),
            pl.BlockSpec((R * OCT, 3 * OCT * F), lambda i: (0, 0)),
        ],
        out_specs=pl.BlockSpec((1, G, OCT * F), lambda i: (i, 0, 0)),
        out_shape=jax.ShapeDtypeStruct((NOCT, G, OCT * F), jnp.float32),
        scratch_shapes=[
            pltpu.VMEM((3, N * N, OCT * F), jnp.float32),
            pltpu.VMEM((3, N, OCT * F), jnp.float32),
        ],
        compiler_params=pltpu.CompilerParams(
            dimension_semantics=("parallel",),
        ),
    )(xab, dg8, w8, wd8)


def kernel(inputs, filters):
    idx = jnp.arange(N)
    in_t = jnp.swapaxes(inputs, 1, 2)
    comp = jnp.concatenate([inputs, in_t], axis=-1)
    xab = (
        comp.reshape(NOCT, OCT, N * N, 2 * R)
        .transpose(0, 2, 3, 1)
        .reshape(NOCT, N * N, 2 * R * OCT)
    )
    diag = inputs[:, idx, idx, :]
    dg8 = (
        diag.reshape(NOCT, OCT, N, R)
        .transpose(0, 2, 1, 3)
        .reshape(NOCT, N, OCT * R)
    )

    def fpq(p, q):
        return filters[:, p, q, :].T

    eye = jnp.eye(OCT, dtype=jnp.float32)
    wuv = jnp.stack([fpq(0, 1), fpq(0, 2), fpq(1, 2)], axis=1)
    wvu = jnp.stack([fpq(1, 0), fpq(2, 0), fpq(2, 1)], axis=1)
    wa3 = jnp.concatenate([wuv, wvu], axis=0)
    w8 = jnp.einsum("kcf,bj->kbcjf", wa3, eye).reshape(2 * R * OCT, 3 * OCT * F)
    wdg = jnp.stack([fpq(0, 0), fpq(1, 1), fpq(2, 2)], axis=1)
    wd8 = jnp.einsum("rcf,bj->brcjf", wdg, eye).reshape(R * OCT, 3 * OCT * F)

    res = _fused_tc(xab, dg8, w8, wd8)
    return res.reshape(NOCT, G, OCT, F).transpose(0, 2, 1, 3).reshape(B, G, F)


# speedup vs baseline: 2.8703x; 1.6402x over previous
import numpy as np

import jax
import jax.numpy as jnp
from jax.experimental import pallas as pl
from jax.experimental.pallas import tpu as pltpu

B = 64
N = 32
R = 16
F = 16
G = 4960

OCT = 8
NOCT = B // OCT


def _perm_matrix():
    p = np.zeros((N * N, N * N), dtype=np.float32)
    src = (np.arange(N * N) % N) * N + np.arange(N * N) // N
    p[np.arange(N * N), src] = 1.0
    return p


def _diag_sel_matrix():
    s = np.zeros((N, N * N), dtype=np.float32)
    s[np.arange(N), np.arange(N) * (N + 1)] = 1.0
    return s


_P = _perm_matrix()
_S = _diag_sel_matrix()


def _fused_body(x_ref, p_ref, s_ref, w1_ref, w2_ref, wd_ref, o_ref,
                scr_ref, dscr_ref):
    xu = x_ref[0]
    xv = jnp.dot(p_ref[...], xu, preferred_element_type=jnp.float32)
    y = (
        jnp.dot(xu, w1_ref[...], preferred_element_type=jnp.float32)
        + jnp.dot(xv, w2_ref[...], preferred_element_type=jnp.float32)
    )
    scr_ref[0] = y[:, 0:128]
    scr_ref[1] = y[:, 128:256]
    scr_ref[2] = y[:, 256:384]
    dsel = jnp.dot(s_ref[...], xu, preferred_element_type=jnp.float32)
    d = jnp.dot(dsel, wd_ref[...], preferred_element_type=jnp.float32)
    dscr_ref[0] = d[:, 0:128]
    dscr_ref[1] = d[:, 128:256]
    dscr_ref[2] = d[:, 256:384]
    off = 0
    for a in range(N - 2):
        for b2 in range(a + 1, N - 1):
            L = (N - 1) - b2
            rowv = scr_ref[0, a * N + b2, :] + dscr_ref[0, a, :] + dscr_ref[1, b2, :]
            s02 = scr_ref[1, pl.ds(a * N + b2 + 1, L), :]
            s12 = scr_ref[2, pl.ds(b2 * N + b2 + 1, L), :]
            d2 = dscr_ref[2, pl.ds(b2 + 1, L), :]
            o_ref[0, pl.ds(off, L), :] = rowv[None, :] + s02 + s12 + d2
            off += L


def _fused_tc(xuv8, pmat, smat, w1, w2, wd):
    return pl.pallas_call(
        _fused_body,
        grid=(NOCT,),
        in_specs=[
            pl.BlockSpec((1, N * N, R * OCT), lambda i: (i, 0, 0)),
            pl.BlockSpec((N * N, N * N), lambda i: (0, 0)),
            pl.BlockSpec((N, N * N), lambda i: (0, 0)),
            pl.BlockSpec((R * OCT, 3 * OCT * F), lambda i: (0, 0)),
            pl.BlockSpec((R * OCT, 3 * OCT * F), lambda i: (0, 0)),
            pl.BlockSpec((R * OCT, 3 * OCT * F), lambda i: (0, 0)),
        ],
        out_specs=pl.BlockSpec((1, G, OCT * F), lambda i: (i, 0, 0)),
        out_shape=jax.ShapeDtypeStruct((NOCT, G, OCT * F), jnp.float32),
        scratch_shapes=[
            pltpu.VMEM((3, N * N, OCT * F), jnp.float32),
            pltpu.VMEM((3, N, OCT * F), jnp.float32),
        ],
        compiler_params=pltpu.CompilerParams(
            dimension_semantics=("parallel",),
        ),
    )(xuv8, pmat, smat, w1, w2, wd)


def kernel(inputs, filters):
    xuv8 = (
        inputs.reshape(NOCT, OCT, N * N, R)
        .transpose(0, 2, 3, 1)
        .reshape(NOCT, N * N, R * OCT)
    )

    def fpq(p, q):
        return filters[:, p, q, :].T

    eye = jnp.eye(OCT, dtype=jnp.float32)
    wuv = jnp.stack([fpq(0, 1), fpq(0, 2), fpq(1, 2)], axis=1)
    wvu = jnp.stack([fpq(1, 0), fpq(2, 0), fpq(2, 1)], axis=1)
    wdg = jnp.stack([fpq(0, 0), fpq(1, 1), fpq(2, 2)], axis=1)
    w1 = jnp.einsum("rcf,bj->rbcjf", wuv, eye).reshape(R * OCT, 3 * OCT * F)
    w2 = jnp.einsum("rcf,bj->rbcjf", wvu, eye).reshape(R * OCT, 3 * OCT * F)
    wd = jnp.einsum("rcf,bj->rbcjf", wdg, eye).reshape(R * OCT, 3 * OCT * F)

    res = _fused_tc(xuv8, jnp.asarray(_P), jnp.asarray(_S), w1, w2, wd)
    return res.reshape(NOCT, G, OCT, F).transpose(0, 2, 1, 3).reshape(B, G, F)


# speedup vs baseline: 3.2554x; 1.1342x over previous
import numpy as np

import jax
import jax.numpy as jnp
from jax.experimental import pallas as pl
from jax.experimental.pallas import tpu as pltpu

B = 64
N = 32
R = 16
F = 16
G = 4960

OCT = 8
NOCT = B // OCT


def _diag_sel_matrix():
    s = np.zeros((N, N * N), dtype=np.float32)
    s[np.arange(N), np.arange(N) * (N + 1)] = 1.0
    return s


_S = _diag_sel_matrix()


def _fused_body(x_ref, s_ref, w1_ref, w2_ref, wd_ref, o_ref,
                scr_ref, dscr_ref):
    xu = x_ref[0]
    xv = jnp.swapaxes(xu.reshape(N, N, R * OCT), 0, 1).reshape(N * N, R * OCT)
    y = (
        jnp.dot(xu, w1_ref[...], preferred_element_type=jnp.float32)
        + jnp.dot(xv, w2_ref[...], preferred_element_type=jnp.float32)
    )
    scr_ref[0] = y[:, 0:128]
    scr_ref[1] = y[:, 128:256]
    scr_ref[2] = y[:, 256:384]
    dsel = jnp.dot(s_ref[...], xu, preferred_element_type=jnp.float32)
    d = jnp.dot(dsel, wd_ref[...], preferred_element_type=jnp.float32)
    dscr_ref[0] = d[:, 0:128]
    dscr_ref[1] = d[:, 128:256]
    dscr_ref[2] = d[:, 256:384]
    off = 0
    for a in range(N - 2):
        for b2 in range(a + 1, N - 1):
            L = (N - 1) - b2
            rowv = scr_ref[0, a * N + b2, :] + dscr_ref[0, a, :] + dscr_ref[1, b2, :]
            s02 = scr_ref[1, pl.ds(a * N + b2 + 1, L), :]
            s12 = scr_ref[2, pl.ds(b2 * N + b2 + 1, L), :]
            d2 = dscr_ref[2, pl.ds(b2 + 1, L), :]
            o_ref[0, pl.ds(off, L), :] = rowv[None, :] + s02 + s12 + d2
            off += L


def _fused_tc(xuv8, smat, w1, w2, wd):
    return pl.pallas_call(
        _fused_body,
        grid=(NOCT,),
        in_specs=[
            pl.BlockSpec((1, N * N, R * OCT), lambda i: (i, 0, 0)),
            pl.BlockSpec((N, N * N), lambda i: (0, 0)),
            pl.BlockSpec((R * OCT, 3 * OCT * F), lambda i: (0, 0)),
            pl.BlockSpec((R * OCT, 3 * OCT * F), lambda i: (0, 0)),
            pl.BlockSpec((R * OCT, 3 * OCT * F), lambda i: (0, 0)),
        ],
        out_specs=pl.BlockSpec((1, G, OCT * F), lambda i: (i, 0, 0)),
        out_shape=jax.ShapeDtypeStruct((NOCT, G, OCT * F), jnp.float32),
        scratch_shapes=[
            pltpu.VMEM((3, N * N, OCT * F), jnp.float32),
            pltpu.VMEM((3, N, OCT * F), jnp.float32),
        ],
        compiler_params=pltpu.CompilerParams(
            dimension_semantics=("parallel",),
        ),
    )(xuv8, smat, w1, w2, wd)


def kernel(inputs, filters):
    xuv8 = (
        inputs.reshape(NOCT, OCT, N * N, R)
        .transpose(0, 2, 3, 1)
        .reshape(NOCT, N * N, R * OCT)
    )

    def fpq(p, q):
        return filters[:, p, q, :].T

    eye = jnp.eye(OCT, dtype=jnp.float32)
    wuv = jnp.stack([fpq(0, 1), fpq(0, 2), fpq(1, 2)], axis=1)
    wvu = jnp.stack([fpq(1, 0), fpq(2, 0), fpq(2, 1)], axis=1)
    wdg = jnp.stack([fpq(0, 0), fpq(1, 1), fpq(2, 2)], axis=1)
    w1 = jnp.einsum("rcf,bj->rbcjf", wuv, eye).reshape(R * OCT, 3 * OCT * F)
    w2 = jnp.einsum("rcf,bj->rbcjf", wvu, eye).reshape(R * OCT, 3 * OCT * F)
    wd = jnp.einsum("rcf,bj->rbcjf", wdg, eye).reshape(R * OCT, 3 * OCT * F)

    res = _fused_tc(xuv8, jnp.asarray(_S), w1, w2, wd)
    return res.reshape(NOCT, G, OCT, F).transpose(0, 2, 1, 3).reshape(B, G, F)
